# baseline (speedup 1.0000x reference)
import jax
import jax.numpy as jnp
from jax import lax
from jax.experimental import pallas as pl
from jax.experimental.pallas import tpu as pltpu
from jax.experimental.pallas import tpu_sc as plsc

_NUM_ENT = 100000
_NUM_REL = 1000
_D = 64
_B = 16384
_NC = 2
_NS = 16
_NW = _NC * _NS
_BPW = _B // _NW
_CH = 64
_NCH = _BPW // _CH
_L = 16

_EPW = _NUM_ENT // _NW
_ECH = 125
_ENCH = _EPW // _ECH
_RPW = 32


def _rsqrt_nr(x):
    i = plsc.bitcast(x, jnp.int32)
    i = jnp.int32(0x5F3759DF) - (i >> 1)
    y = plsc.bitcast(i, jnp.float32)
    for _ in range(3):
        y = y * (1.5 - 0.5 * x * y * y)
    return y


def _sc_body(h_hbm, r_hbm, t_hbm, n_hbm, ent_hbm, relnorm_hbm, rel_hbm, norm_hbm,
             score_hbm, nscore_hbm, spart_hbm, orth_hbm,
             hi, ri, ti, ni,
             hbufs, tbufs, nbufs, rwbufs, s1, s2,
             ebuf2, sbuf, r2b, n2b, srnb, srrb, valb, accb,
             gsems, esem, isem):
    wid = lax.axis_index("s") * _NC + lax.axis_index("c")
    base = wid * _BPW
    idx_cps = [
        pltpu.async_copy(h_hbm.at[pl.ds(base, _BPW)], hi, isem),
        pltpu.async_copy(r_hbm.at[pl.ds(base, _BPW)], ri, isem),
        pltpu.async_copy(t_hbm.at[pl.ds(base, _BPW)], ti, isem),
        pltpu.async_copy(n_hbm.at[pl.ds(base, _BPW)], ni, isem),
    ]
    for cp in idx_cps:
        cp.wait()

    lane0 = lax.iota(jnp.int32, _L) == 0

    ebase = wid * _EPW
    for p in range(3):
        pltpu.async_copy(ent_hbm.at[pl.ds(ebase + p * _ECH, _ECH)],
                         ebuf2.at[pl.ds(p * _ECH, _ECH)], esem)

    def fire(c, b):
        cb = c * _CH
        return [
            pltpu.async_copy(ent_hbm.at[hi.at[pl.ds(cb, _CH)]], hbufs[b],
                             gsems[b]),
            pltpu.async_copy(ent_hbm.at[ti.at[pl.ds(cb, _CH)]], tbufs[b],
                             gsems[b]),
            pltpu.async_copy(ent_hbm.at[ni.at[pl.ds(cb, _CH)]], nbufs[b],
                             gsems[b]),
            pltpu.async_copy(relnorm_hbm.at[ri.at[pl.ds(cb, _CH)]], rwbufs[b],
                             gsems[b]),
        ]

    inflight = {c: fire(c, c) for c in range(3)}
    for c in range(_NCH):
        b = c % 4
        if c + 3 < _NCH:
            inflight[c + 3] = fire(c + 3, (c + 3) % 4)
        for cp in inflight.pop(c):
            cp.wait()
        hbuf, tbuf, nbuf, rwbuf = (hbufs[b], tbufs[b], nbufs[b], rwbufs[b])
        cb = c * _CH

        def row(i, cb=cb, hbuf=hbuf, tbuf=tbuf, nbuf=nbuf, rwbuf=rwbuf):
            w = [rwbuf[i, pl.ds(_D + _L * j, _L)] for j in range(4)]
            hh = [hbuf[i, pl.ds(_L * j, _L)] for j in range(4)]
            tt = [tbuf[i, pl.ds(_L * j, _L)] for j in range(4)]
            nn = [nbuf[i, pl.ds(_L * j, _L)] for j in range(4)]
            rr = [rwbuf[i, pl.ds(_L * j, _L)] for j in range(4)]
            ww = w[0] * w[0] + w[1] * w[1] + w[2] * w[2] + w[3] * w[3]
            wh = w[0] * hh[0] + w[1] * hh[1] + w[2] * hh[2] + w[3] * hh[3]
            wt = w[0] * tt[0] + w[1] * tt[1] + w[2] * tt[2] + w[3] * tt[3]
            wn = w[0] * nn[0] + w[1] * nn[1] + w[2] * nn[2] + w[3] * nn[3]
            sww = jnp.broadcast_to(jnp.sum(ww), (_L,))
            swh = jnp.broadcast_to(jnp.sum(wh), (_L,))
            swt = jnp.broadcast_to(jnp.sum(wt), (_L,))
            swn = jnp.broadcast_to(jnp.sum(wn), (_L,))
            inv = 1.0 / jnp.maximum(sww, jnp.float32(1e-24))
            alpha = (swh - swt) * inv
            beta = (swh - swn) * inv
            acc1 = None
            acc2 = None
            for j in range(4):
                d = hh[j] + rr[j]
                e1 = d - tt[j] - alpha * w[j]
                e2 = d - nn[j] - beta * w[j]
                acc1 = e1 * e1 if acc1 is None else acc1 + e1 * e1
                acc2 = e2 * e2 if acc2 is None else acc2 + e2 * e2
            pos = jnp.broadcast_to(cb + i, (_L,))
            plsc.store_scatter(s1, [pos], jnp.broadcast_to(jnp.sum(acc1), (_L,)),
                               mask=lane0)
            plsc.store_scatter(s2, [pos], jnp.broadcast_to(jnp.sum(acc2), (_L,)),
                               mask=lane0)
        plsc.parallel_loop(0, _CH, 1, unroll=4)(row)

    def sqrt_pass(k):
        v1 = jnp.maximum(s1[pl.ds(_L * k, _L)], jnp.float32(1e-30))
        v2 = jnp.maximum(s2[pl.ds(_L * k, _L)], jnp.float32(1e-30))
        s1[pl.ds(_L * k, _L)] = v1 * _rsqrt_nr(v1)
        s2[pl.ds(_L * k, _L)] = v2 * _rsqrt_nr(v2)

    plsc.parallel_loop(0, _BPW // _L, 1, unroll=2)(sqrt_pass)

    out1 = pltpu.async_copy(s1, score_hbm.at[pl.ds(base, _BPW)], isem)
    out2 = pltpu.async_copy(s2, nscore_hbm.at[pl.ds(base, _BPW)], isem)

    ones = jnp.ones((_L,), jnp.float32)

    def ent_chunk(c, acc):
        off = (c % 4) * _ECH
        row0 = ebase + c * _ECH

        @pl.when(c + 3 < _ENCH)
        def _():
            noff = ((c + 3) % 4) * _ECH
            pltpu.async_copy(ent_hbm.at[pl.ds(row0 + 3 * _ECH, _ECH)],
                             ebuf2.at[pl.ds(noff, _ECH)], esem)

        pltpu.make_async_copy(ent_hbm.at[pl.ds(row0, _ECH)],
                              ebuf2.at[pl.ds(off, _ECH)], esem).wait()
        sbuf[pl.ds(112, _L)] = ones

        def erow(i, off=off):
            x = [ebuf2[off + i, pl.ds(_L * j, _L)] for j in range(4)]
            sq = x[0] * x[0] + x[1] * x[1] + x[2] * x[2] + x[3] * x[3]
            pos = jnp.broadcast_to(i, (_L,))
            plsc.store_scatter(sbuf, [pos],
                               jnp.broadcast_to(jnp.sum(sq), (_L,)),
                               mask=lane0)

        plsc.parallel_loop(0, _ECH, 1, unroll=4)(erow)
        for k in range(8):
            m = jnp.maximum(sbuf[pl.ds(_L * k, _L)], jnp.float32(1.0))
            acc = acc + jnp.maximum(m * _rsqrt_nr(m) - 1.0, jnp.float32(0.0))
        return acc

    acc = lax.fori_loop(0, _ENCH, ent_chunk, jnp.zeros((_L,), jnp.float32))
    accb[pl.ds(0, _L)] = acc
    pltpu.sync_copy(accb, spart_hbm.at[wid])

    rstart = jnp.minimum(wid * _RPW, jnp.int32(_NUM_REL - _RPW))
    pltpu.sync_copy(rel_hbm.at[pl.ds(rstart, _RPW)], r2b)
    pltpu.sync_copy(norm_hbm.at[pl.ds(rstart, _RPW)], n2b)

    def rrow(i):
        rv = [r2b[i, pl.ds(_L * j, _L)] for j in range(4)]
        nv = [n2b[i, pl.ds(_L * j, _L)] for j in range(4)]
        rn = rv[0] * nv[0] + rv[1] * nv[1] + rv[2] * nv[2] + rv[3] * nv[3]
        r2 = rv[0] * rv[0] + rv[1] * rv[1] + rv[2] * rv[2] + rv[3] * rv[3]
        pos = jnp.broadcast_to(i, (_L,))
        plsc.store_scatter(srnb, [pos],
                           jnp.broadcast_to(jnp.sum(rn), (_L,)), mask=lane0)
        plsc.store_scatter(srrb, [pos],
                           jnp.broadcast_to(jnp.sum(r2), (_L,)), mask=lane0)

    plsc.parallel_loop(0, _RPW, 1, unroll=4)(rrow)
    for k in range(_RPW // _L):
        valb[pl.ds(_L * k, _L)] = (srnb[pl.ds(_L * k, _L)]
                                   * _rsqrt_nr(srrb[pl.ds(_L * k, _L)]))

    @pl.when(wid < _NW - 1)
    def _():
        pltpu.sync_copy(valb, orth_hbm.at[pl.ds(wid * _RPW, _RPW)])

    @pl.when(wid == _NW - 1)
    def _():
        pltpu.sync_copy(valb.at[pl.ds(24, 8)],
                        orth_hbm.at[pl.ds(_NUM_REL - 8, 8)])

    out1.wait()
    out2.wait()


def kernel(h, basic_r, t, neg_idx, ent_table, rel_table, norm_table):
    mesh = plsc.VectorSubcoreMesh(core_axis_name="c", subcore_axis_name="s")
    rowbuf = pltpu.VMEM((_CH, _D), jnp.float32)
    sc_call = pl.kernel(
        _sc_body,
        out_type=(
            jax.ShapeDtypeStruct((_B,), jnp.float32),
            jax.ShapeDtypeStruct((_B,), jnp.float32),
            jax.ShapeDtypeStruct((_NW, _L), jnp.float32),
            jax.ShapeDtypeStruct((_NUM_REL,), jnp.float32),
        ),
        mesh=mesh,
        compiler_params=pltpu.CompilerParams(
            needs_layout_passes=False, use_tc_tiling_on_sc=False,
            disable_bounds_checks=True, skip_device_barrier=True),
        scratch_types=[
            pltpu.VMEM((_BPW,), jnp.int32),
            pltpu.VMEM((_BPW,), jnp.int32),
            pltpu.VMEM((_BPW,), jnp.int32),
            pltpu.VMEM((_BPW,), jnp.int32),
            (rowbuf, rowbuf, rowbuf, rowbuf),
            (rowbuf, rowbuf, rowbuf, rowbuf),
            (rowbuf, rowbuf, rowbuf, rowbuf),
            (pltpu.VMEM((_CH, 2 * _D), jnp.float32),
             pltpu.VMEM((_CH, 2 * _D), jnp.float32),
             pltpu.VMEM((_CH, 2 * _D), jnp.float32),
             pltpu.VMEM((_CH, 2 * _D), jnp.float32)),
            pltpu.VMEM((_BPW,), jnp.float32),
            pltpu.VMEM((_BPW,), jnp.float32),
            pltpu.VMEM((4 * _ECH, _D), jnp.float32),
            pltpu.VMEM((128,), jnp.float32),
            pltpu.VMEM((_RPW, _D), jnp.float32),
            pltpu.VMEM((_RPW, _D), jnp.float32),
            pltpu.VMEM((_RPW,), jnp.float32),
            pltpu.VMEM((_RPW,), jnp.float32),
            pltpu.VMEM((_RPW,), jnp.float32),
            pltpu.VMEM((_L,), jnp.float32),
            (pltpu.SemaphoreType.DMA, pltpu.SemaphoreType.DMA,
             pltpu.SemaphoreType.DMA, pltpu.SemaphoreType.DMA),
            pltpu.SemaphoreType.DMA,
            pltpu.SemaphoreType.DMA,
        ],
    )
    relnorm = jnp.concatenate([rel_table, norm_table], axis=1)
    score, neg_score, sparts, orth = sc_call(
        h, basic_r, t, neg_idx, ent_table, relnorm, rel_table, norm_table)

    scale_loss = jnp.sum(sparts) / _NUM_ENT
    return (score, neg_score, scale_loss, orth)


# speedup vs baseline: 1.0095x; 1.0095x over previous
import jax
import jax.numpy as jnp
from jax import lax
from jax.experimental import pallas as pl
from jax.experimental.pallas import tpu as pltpu
from jax.experimental.pallas import tpu_sc as plsc

_NUM_ENT = 100000
_NUM_REL = 1000
_D = 64
_B = 16384
_NC = 2
_NS = 16
_NW = _NC * _NS
_BPW = _B // _NW
_CH = 128
_NCH = _BPW // _CH
_L = 16

_EPW = _NUM_ENT // _NW
_ECH = 125
_ENCH = _EPW // _ECH
_RPW = 32


def _rsqrt_nr(x):
    i = plsc.bitcast(x, jnp.int32)
    i = jnp.int32(0x5F3759DF) - (i >> 1)
    y = plsc.bitcast(i, jnp.float32)
    for _ in range(3):
        y = y * (1.5 - 0.5 * x * y * y)
    return y


def _sc_body(h_hbm, r_hbm, t_hbm, n_hbm, ent_hbm, relnorm_hbm, rel_hbm, norm_hbm,
             score_hbm, nscore_hbm, spart_hbm, orth_hbm,
             hi, ri, ti, ni,
             hbufs, tbufs, nbufs, rwbufs, s1, s2,
             ebuf2, sbuf, r2b, n2b, srnb, srrb, valb, accb,
             gsems, esem, isem):
    wid = lax.axis_index("s") * _NC + lax.axis_index("c")
    base = wid * _BPW
    idx_cps = [
        pltpu.async_copy(h_hbm.at[pl.ds(base, _BPW)], hi, isem),
        pltpu.async_copy(r_hbm.at[pl.ds(base, _BPW)], ri, isem),
        pltpu.async_copy(t_hbm.at[pl.ds(base, _BPW)], ti, isem),
        pltpu.async_copy(n_hbm.at[pl.ds(base, _BPW)], ni, isem),
    ]
    for cp in idx_cps:
        cp.wait()

    lane0 = lax.iota(jnp.int32, _L) == 0

    ebase = wid * _EPW
    for p in range(3):
        pltpu.async_copy(ent_hbm.at[pl.ds(ebase + p * _ECH, _ECH)],
                         ebuf2.at[pl.ds(p * _ECH, _ECH)], esem)

    def fire(c, b):
        cb = c * _CH
        return [
            pltpu.async_copy(ent_hbm.at[hi.at[pl.ds(cb, _CH)]], hbufs[b],
                             gsems[b]),
            pltpu.async_copy(ent_hbm.at[ti.at[pl.ds(cb, _CH)]], tbufs[b],
                             gsems[b]),
            pltpu.async_copy(ent_hbm.at[ni.at[pl.ds(cb, _CH)]], nbufs[b],
                             gsems[b]),
            pltpu.async_copy(relnorm_hbm.at[ri.at[pl.ds(cb, _CH)]], rwbufs[b],
                             gsems[b]),
        ]

    inflight = {0: fire(0, 0)}
    for c in range(_NCH):
        b = c % 2
        if c + 1 < _NCH:
            inflight[c + 1] = fire(c + 1, (c + 1) % 2)
        for cp in inflight.pop(c):
            cp.wait()
        hbuf, tbuf, nbuf, rwbuf = (hbufs[b], tbufs[b], nbufs[b], rwbufs[b])
        cb = c * _CH

        def row(i, cb=cb, hbuf=hbuf, tbuf=tbuf, nbuf=nbuf, rwbuf=rwbuf):
            w = [rwbuf[i, pl.ds(_D + _L * j, _L)] for j in range(4)]
            hh = [hbuf[i, pl.ds(_L * j, _L)] for j in range(4)]
            tt = [tbuf[i, pl.ds(_L * j, _L)] for j in range(4)]
            nn = [nbuf[i, pl.ds(_L * j, _L)] for j in range(4)]
            rr = [rwbuf[i, pl.ds(_L * j, _L)] for j in range(4)]
            ww = w[0] * w[0] + w[1] * w[1] + w[2] * w[2] + w[3] * w[3]
            wh = w[0] * hh[0] + w[1] * hh[1] + w[2] * hh[2] + w[3] * hh[3]
            wt = w[0] * tt[0] + w[1] * tt[1] + w[2] * tt[2] + w[3] * tt[3]
            wn = w[0] * nn[0] + w[1] * nn[1] + w[2] * nn[2] + w[3] * nn[3]
            sww = jnp.broadcast_to(jnp.sum(ww), (_L,))
            swh = jnp.broadcast_to(jnp.sum(wh), (_L,))
            swt = jnp.broadcast_to(jnp.sum(wt), (_L,))
            swn = jnp.broadcast_to(jnp.sum(wn), (_L,))
            inv = 1.0 / jnp.maximum(sww, jnp.float32(1e-24))
            alpha = (swh - swt) * inv
            beta = (swh - swn) * inv
            acc1 = None
            acc2 = None
            for j in range(4):
                d = hh[j] + rr[j]
                e1 = d - tt[j] - alpha * w[j]
                e2 = d - nn[j] - beta * w[j]
                acc1 = e1 * e1 if acc1 is None else acc1 + e1 * e1
                acc2 = e2 * e2 if acc2 is None else acc2 + e2 * e2
            pos = jnp.broadcast_to(cb + i, (_L,))
            plsc.store_scatter(s1, [pos], jnp.broadcast_to(jnp.sum(acc1), (_L,)),
                               mask=lane0)
            plsc.store_scatter(s2, [pos], jnp.broadcast_to(jnp.sum(acc2), (_L,)),
                               mask=lane0)
        plsc.parallel_loop(0, _CH, 1, unroll=4)(row)

    def sqrt_pass(k):
        v1 = jnp.maximum(s1[pl.ds(_L * k, _L)], jnp.float32(1e-30))
        v2 = jnp.maximum(s2[pl.ds(_L * k, _L)], jnp.float32(1e-30))
        s1[pl.ds(_L * k, _L)] = v1 * _rsqrt_nr(v1)
        s2[pl.ds(_L * k, _L)] = v2 * _rsqrt_nr(v2)

    plsc.parallel_loop(0, _BPW // _L, 1, unroll=2)(sqrt_pass)

    out1 = pltpu.async_copy(s1, score_hbm.at[pl.ds(base, _BPW)], isem)
    out2 = pltpu.async_copy(s2, nscore_hbm.at[pl.ds(base, _BPW)], isem)

    ones = jnp.ones((_L,), jnp.float32)

    def ent_chunk(c, acc):
        off = (c % 4) * _ECH
        row0 = ebase + c * _ECH

        @pl.when(c + 3 < _ENCH)
        def _():
            noff = ((c + 3) % 4) * _ECH
            pltpu.async_copy(ent_hbm.at[pl.ds(row0 + 3 * _ECH, _ECH)],
                             ebuf2.at[pl.ds(noff, _ECH)], esem)

        pltpu.make_async_copy(ent_hbm.at[pl.ds(row0, _ECH)],
                              ebuf2.at[pl.ds(off, _ECH)], esem).wait()
        sbuf[pl.ds(112, _L)] = ones

        def erow(i, off=off):
            x = [ebuf2[off + i, pl.ds(_L * j, _L)] for j in range(4)]
            sq = x[0] * x[0] + x[1] * x[1] + x[2] * x[2] + x[3] * x[3]
            pos = jnp.broadcast_to(i, (_L,))
            plsc.store_scatter(sbuf, [pos],
                               jnp.broadcast_to(jnp.sum(sq), (_L,)),
                               mask=lane0)

        plsc.parallel_loop(0, _ECH, 1, unroll=4)(erow)
        for k in range(8):
            m = jnp.maximum(sbuf[pl.ds(_L * k, _L)], jnp.float32(1.0))
            acc = acc + jnp.maximum(m * _rsqrt_nr(m) - 1.0, jnp.float32(0.0))
        return acc

    acc = lax.fori_loop(0, _ENCH, ent_chunk, jnp.zeros((_L,), jnp.float32))
    accb[pl.ds(0, _L)] = acc
    pltpu.sync_copy(accb, spart_hbm.at[wid])

    rstart = jnp.minimum(wid * _RPW, jnp.int32(_NUM_REL - _RPW))
    pltpu.sync_copy(rel_hbm.at[pl.ds(rstart, _RPW)], r2b)
    pltpu.sync_copy(norm_hbm.at[pl.ds(rstart, _RPW)], n2b)

    def rrow(i):
        rv = [r2b[i, pl.ds(_L * j, _L)] for j in range(4)]
        nv = [n2b[i, pl.ds(_L * j, _L)] for j in range(4)]
        rn = rv[0] * nv[0] + rv[1] * nv[1] + rv[2] * nv[2] + rv[3] * nv[3]
        r2 = rv[0] * rv[0] + rv[1] * rv[1] + rv[2] * rv[2] + rv[3] * rv[3]
        pos = jnp.broadcast_to(i, (_L,))
        plsc.store_scatter(srnb, [pos],
                           jnp.broadcast_to(jnp.sum(rn), (_L,)), mask=lane0)
        plsc.store_scatter(srrb, [pos],
                           jnp.broadcast_to(jnp.sum(r2), (_L,)), mask=lane0)

    plsc.parallel_loop(0, _RPW, 1, unroll=4)(rrow)
    for k in range(_RPW // _L):
        valb[pl.ds(_L * k, _L)] = (srnb[pl.ds(_L * k, _L)]
                                   * _rsqrt_nr(srrb[pl.ds(_L * k, _L)]))

    @pl.when(wid < _NW - 1)
    def _():
        pltpu.sync_copy(valb, orth_hbm.at[pl.ds(wid * _RPW, _RPW)])

    @pl.when(wid == _NW - 1)
    def _():
        pltpu.sync_copy(valb.at[pl.ds(24, 8)],
                        orth_hbm.at[pl.ds(_NUM_REL - 8, 8)])

    out1.wait()
    out2.wait()


def kernel(h, basic_r, t, neg_idx, ent_table, rel_table, norm_table):
    mesh = plsc.VectorSubcoreMesh(core_axis_name="c", subcore_axis_name="s")
    rowbuf = pltpu.VMEM((_CH, _D), jnp.float32)
    sc_call = pl.kernel(
        _sc_body,
        out_type=(
            jax.ShapeDtypeStruct((_B,), jnp.float32),
            jax.ShapeDtypeStruct((_B,), jnp.float32),
            jax.ShapeDtypeStruct((_NW, _L), jnp.float32),
            jax.ShapeDtypeStruct((_NUM_REL,), jnp.float32),
        ),
        mesh=mesh,
        compiler_params=pltpu.CompilerParams(
            needs_layout_passes=False, use_tc_tiling_on_sc=False,
            disable_bounds_checks=True, skip_device_barrier=True),
        scratch_types=[
            pltpu.VMEM((_BPW,), jnp.int32),
            pltpu.VMEM((_BPW,), jnp.int32),
            pltpu.VMEM((_BPW,), jnp.int32),
            pltpu.VMEM((_BPW,), jnp.int32),
            (rowbuf, rowbuf),
            (rowbuf, rowbuf),
            (rowbuf, rowbuf),
            (pltpu.VMEM((_CH, 2 * _D), jnp.float32),
             pltpu.VMEM((_CH, 2 * _D), jnp.float32)),
            pltpu.VMEM((_BPW,), jnp.float32),
            pltpu.VMEM((_BPW,), jnp.float32),
            pltpu.VMEM((4 * _ECH, _D), jnp.float32),
            pltpu.VMEM((128,), jnp.float32),
            pltpu.VMEM((_RPW, _D), jnp.float32),
            pltpu.VMEM((_RPW, _D), jnp.float32),
            pltpu.VMEM((_RPW,), jnp.float32),
            pltpu.VMEM((_RPW,), jnp.float32),
            pltpu.VMEM((_RPW,), jnp.float32),
            pltpu.VMEM((_L,), jnp.float32),
            (pltpu.SemaphoreType.DMA, pltpu.SemaphoreType.DMA),
            pltpu.SemaphoreType.DMA,
            pltpu.SemaphoreType.DMA,
        ],
    )
    relnorm = jnp.concatenate([rel_table, norm_table], axis=1)
    score, neg_score, sparts, orth = sc_call(
        h, basic_r, t, neg_idx, ent_table, relnorm, rel_table, norm_table)

    scale_loss = jnp.sum(sparts) / _NUM_ENT
    return (score, neg_score, scale_loss, orth)
